# Initial kernel scaffold; baseline (speedup 1.0000x reference)
import jax
import jax.numpy as jnp
from jax.experimental import pallas as pl


def kernel(x, router_w, w1, w2):
    raise NotImplementedError("write your pallas kernel here")


# speedup vs baseline: 2.4589x; 2.4589x over previous
import functools

import jax
import jax.numpy as jnp
from jax import lax
from jax.experimental import pallas as pl
from jax.experimental.pallas import tpu as pltpu
from jax.experimental.pallas import tpu_sc as plsc

TOPK = 2
BM = 512
NLANE = 128
NEG = -1e30

SC_CORES = 2
SC_SUBCORES = 16
NW = SC_CORES * SC_SUBCORES


def _gelu_exact(v):
    return v * 0.5 * (1.0 + lax.erf(v * 0.7071067811865476))


def _router_body(x_ref, rw_ref, idx_ref, w_ref, *, n_exp):
    logits = lax.dot_general(
        x_ref[...], rw_ref[...], (((1,), (1,)), ((), ())),
        preferred_element_type=jnp.float32)
    lane = lax.broadcasted_iota(jnp.int32, logits.shape, 1)
    l = jnp.where(lane < n_exp, logits, NEG)
    m1 = jnp.max(l, axis=1, keepdims=True)
    i1 = jnp.min(jnp.where(l == m1, lane, NLANE - 1), axis=1, keepdims=True)
    l2 = jnp.where(lane == i1, NEG, l)
    m2 = jnp.max(l2, axis=1, keepdims=True)
    i2 = jnp.min(jnp.where(l2 == m2, lane, NLANE - 1), axis=1, keepdims=True)
    s = jnp.exp(m2 - m1)
    w0 = 1.0 / (1.0 + s)
    w1 = s / (1.0 + s)
    idx_ref[...] = jnp.where(lane == 0, i1, i2)
    w_ref[...] = jnp.where(lane == 0, w0, w1)


def _router(x_flat, router_w):
    n, d = x_flat.shape
    n_exp = router_w.shape[0]
    rw_pad = jnp.zeros((NLANE, d), jnp.float32).at[:n_exp].set(router_w)
    bm = 512
    idx, w = pl.pallas_call(
        functools.partial(_router_body, n_exp=n_exp),
        grid=(n // bm,),
        in_specs=[
            pl.BlockSpec((bm, d), lambda i: (i, 0)),
            pl.BlockSpec((NLANE, d), lambda i: (0, 0)),
        ],
        out_specs=[
            pl.BlockSpec((bm, NLANE), lambda i: (i, 0)),
            pl.BlockSpec((bm, NLANE), lambda i: (i, 0)),
        ],
        out_shape=[
            jax.ShapeDtypeStruct((n, NLANE), jnp.int32),
            jax.ShapeDtypeStruct((n, NLANE), jnp.float32),
        ],
    )(x_flat, rw_pad)
    return idx[:, 0], idx[:, 1], w[:, 0], w[:, 1]


def _bookkeeping(idx0, idx1, w0, w1, n_exp, nb):
    p = idx0.shape[0] * TOPK
    e = jnp.stack([idx0, idx1], axis=1).reshape(-1)
    onehot = (e[:, None] == jnp.arange(n_exp)[None, :]).astype(jnp.int32)
    cum = jnp.cumsum(onehot, axis=0)
    rank = jnp.sum(onehot * cum, axis=1) - 1
    counts = cum[-1]
    padded_counts = -(-counts // BM) * BM
    padded_end = jnp.cumsum(padded_counts)
    padded_start = padded_end - padded_counts
    dest = jnp.sum(onehot * padded_start[None, :], axis=1) + rank
    token_src = jnp.arange(p, dtype=jnp.int32) // TOPK
    wpair = jnp.stack([w0, w1], axis=1).reshape(-1)
    bstart = jnp.arange(nb, dtype=jnp.int32) * BM
    block_expert = jnp.minimum(
        jnp.sum((bstart[:, None] >= padded_end[None, :]).astype(jnp.int32),
                axis=1), n_exp - 1).astype(jnp.int32)
    return dest.astype(jnp.int32), token_src, wpair, block_expert


def _sc_dispatch(x_flat, token_src, dest, wpair, np_rows):
    n, d = x_flat.shape
    p = token_src.shape[0]
    per_w = p // NW
    chunk = 64
    nch = per_w // chunk
    mesh = plsc.VectorSubcoreMesh(
        core_axis_name="c", subcore_axis_name="s",
        num_cores=SC_CORES, num_subcores=SC_SUBCORES)

    @functools.partial(
        pl.kernel,
        out_type=[
            jax.ShapeDtypeStruct((np_rows, d), jnp.float32),
            jax.ShapeDtypeStruct((np_rows,), jnp.float32),
        ],
        mesh=mesh,
        scratch_types=[
            pltpu.VMEM((chunk,), jnp.int32),
            pltpu.VMEM((chunk,), jnp.int32),
            pltpu.VMEM((chunk, d), jnp.float32),
            pltpu.VMEM((chunk,), jnp.float32),
            pltpu.SemaphoreType.DMA,
            pltpu.SemaphoreType.DMA,
        ],
    )
    def dispatch(x_hbm, tok_hbm, dest_hbm, wpair_hbm, xg_hbm, wpad_hbm,
                 tok_v, dst_v, rows_v, wrow_v, sem_g, sem_s):
        wid = lax.axis_index("s") * SC_CORES + lax.axis_index("c")
        base0 = wid * per_w
        for c in range(nch):
            base = base0 + c * chunk
            pltpu.sync_copy(tok_hbm.at[pl.ds(base, chunk)], tok_v)
            pltpu.sync_copy(dest_hbm.at[pl.ds(base, chunk)], dst_v)
            pltpu.sync_copy(wpair_hbm.at[pl.ds(base, chunk)], wrow_v)
            pltpu.async_copy(x_hbm.at[tok_v], rows_v, sem_g).wait()
            pltpu.async_copy(rows_v, xg_hbm.at[dst_v], sem_s).wait()
            pltpu.async_copy(wrow_v, wpad_hbm.at[dst_v], sem_s).wait()

    return dispatch(x_flat, token_src, dest, wpair)


def _sc_combine(y, dest0, dest1):
    np_rows, d = y.shape
    n = dest0.shape[0]
    per_w = n // NW
    chunk = 32
    nch = per_w // chunk
    mesh = plsc.VectorSubcoreMesh(
        core_axis_name="c", subcore_axis_name="s",
        num_cores=SC_CORES, num_subcores=SC_SUBCORES)

    @functools.partial(
        pl.kernel,
        out_type=jax.ShapeDtypeStruct((n, d), jnp.float32),
        mesh=mesh,
        scratch_types=[
            pltpu.VMEM((chunk,), jnp.int32),
            pltpu.VMEM((chunk,), jnp.int32),
            pltpu.VMEM((chunk, d), jnp.float32),
            pltpu.VMEM((chunk, d), jnp.float32),
            pltpu.SemaphoreType.DMA,
        ],
    )
    def combine(y_hbm, d0_hbm, d1_hbm, out_hbm,
                d0_v, d1_v, rows_a, rows_b, sem):
        wid = lax.axis_index("s") * SC_CORES + lax.axis_index("c")
        base0 = wid * per_w
        nslice = d // 16
        for c in range(nch):
            base = base0 + c * chunk
            pltpu.sync_copy(d0_hbm.at[pl.ds(base, chunk)], d0_v)
            pltpu.sync_copy(d1_hbm.at[pl.ds(base, chunk)], d1_v)
            pltpu.async_copy(y_hbm.at[d0_v], rows_a, sem).wait()
            pltpu.async_copy(y_hbm.at[d1_v], rows_b, sem).wait()
            for r in range(chunk):
                def add_body(v, _):
                    off = v * 16
                    rows_a[r, pl.ds(off, 16)] = (
                        rows_a[r, pl.ds(off, 16)] + rows_b[r, pl.ds(off, 16)])
                    return _
                lax.fori_loop(0, nslice, add_body, 0, unroll=4)
            pltpu.sync_copy(rows_a, out_hbm.at[pl.ds(base, chunk)])

    return combine(y, dest0, dest1)


def _ffn_body(se_ref, xg_ref, w1_ref, w2_ref, wp_ref, y_ref, *, nj):
    j = pl.program_id(1)
    xb = xg_ref[...]
    h = lax.dot_general(xb, w1_ref[0], (((1,), (1,)), ((), ())),
                        precision=lax.Precision.HIGHEST,
                        preferred_element_type=jnp.float32)
    h = _gelu_exact(h)
    part = lax.dot_general(h, w2_ref[0], (((1,), (1,)), ((), ())),
                           precision=lax.Precision.HIGHEST,
                           preferred_element_type=jnp.float32)

    @pl.when(j == 0)
    def _():
        y_ref[...] = part

    @pl.when(j > 0)
    def _():
        y_ref[...] += part

    @pl.when(j == nj - 1)
    def _():
        y_ref[...] *= wp_ref[0, 0, :][:, None]


def _ffn(xg, wpad, w1, w2, block_expert):
    np_rows, d = xg.shape
    n_exp, ff, _ = w1.shape
    nb = np_rows // BM
    nj = 4
    bf = ff // nj
    wpad3 = wpad.reshape(nb, 1, BM)
    grid_spec = pltpu.PrefetchScalarGridSpec(
        num_scalar_prefetch=1,
        grid=(nb, nj),
        in_specs=[
            pl.BlockSpec((BM, d), lambda i, j, se: (i, 0)),
            pl.BlockSpec((1, bf, d), lambda i, j, se: (se[i], j, 0)),
            pl.BlockSpec((1, d, bf), lambda i, j, se: (se[i], 0, j)),
            pl.BlockSpec((1, 1, BM), lambda i, j, se: (i, 0, 0)),
        ],
        out_specs=pl.BlockSpec((BM, d), lambda i, j, se: (i, 0)),
    )
    return pl.pallas_call(
        functools.partial(_ffn_body, nj=nj),
        grid_spec=grid_spec,
        out_shape=jax.ShapeDtypeStruct((np_rows, d), jnp.float32),
    )(block_expert, xg, w1, w2, wpad3)


def kernel(x, router_w, w1, w2):
    b, t, d = x.shape
    n_exp, ff, _ = w1.shape
    n = b * t
    x_flat = x.reshape(n, d)
    nb = (n * TOPK) // BM + n_exp
    np_rows = nb * BM

    idx0, idx1, gw0, gw1 = _router(x_flat, router_w)
    dest, token_src, wpair, block_expert = _bookkeeping(
        idx0, idx1, gw0, gw1, n_exp, nb)
    xg, wpad = _sc_dispatch(x_flat, token_src, dest, wpair, np_rows)
    y = _ffn(xg, wpad, w1, w2, block_expert)
    dest0 = dest[0::TOPK]
    dest1 = dest[1::TOPK]
    out = _sc_combine(y, dest0, dest1)
    return out.reshape(b, t, d)


# speedup vs baseline: 7.4021x; 3.0103x over previous
import functools

import jax
import jax.numpy as jnp
from jax import lax
from jax.experimental import pallas as pl
from jax.experimental.pallas import tpu as pltpu
from jax.experimental.pallas import tpu_sc as plsc

TOPK = 2
BM = 512
NLANE = 128
NEG = -1e30

SC_CORES = 2
SC_SUBCORES = 16
NW = SC_CORES * SC_SUBCORES


def _gelu_exact(v):
    return v * 0.5 * (1.0 + lax.erf(v * 0.7071067811865476))


def _router_body(x_ref, rw_ref, idx_ref, w_ref, *, n_exp):
    logits = lax.dot_general(
        x_ref[...], rw_ref[...], (((1,), (1,)), ((), ())),
        preferred_element_type=jnp.float32)
    lane = lax.broadcasted_iota(jnp.int32, logits.shape, 1)
    l = jnp.where(lane < n_exp, logits, NEG)
    m1 = jnp.max(l, axis=1, keepdims=True)
    i1 = jnp.min(jnp.where(l == m1, lane, NLANE - 1), axis=1, keepdims=True)
    l2 = jnp.where(lane == i1, NEG, l)
    m2 = jnp.max(l2, axis=1, keepdims=True)
    i2 = jnp.min(jnp.where(l2 == m2, lane, NLANE - 1), axis=1, keepdims=True)
    s = jnp.exp(m2 - m1)
    w0 = 1.0 / (1.0 + s)
    w1 = s / (1.0 + s)
    idx_ref[...] = jnp.where(lane == 0, i1, i2)
    w_ref[...] = jnp.where(lane == 0, w0, w1)


def _router(x_flat, router_w):
    n, d = x_flat.shape
    n_exp = router_w.shape[0]
    rw_pad = jnp.zeros((NLANE, d), jnp.float32).at[:n_exp].set(router_w)
    bm = 512
    idx, w = pl.pallas_call(
        functools.partial(_router_body, n_exp=n_exp),
        grid=(n // bm,),
        in_specs=[
            pl.BlockSpec((bm, d), lambda i: (i, 0)),
            pl.BlockSpec((NLANE, d), lambda i: (0, 0)),
        ],
        out_specs=[
            pl.BlockSpec((bm, NLANE), lambda i: (i, 0)),
            pl.BlockSpec((bm, NLANE), lambda i: (i, 0)),
        ],
        out_shape=[
            jax.ShapeDtypeStruct((n, NLANE), jnp.int32),
            jax.ShapeDtypeStruct((n, NLANE), jnp.float32),
        ],
    )(x_flat, rw_pad)
    return idx[:, 0], idx[:, 1], w[:, 0], w[:, 1]


def _bookkeeping(idx0, idx1, w0, w1, n_exp, nb):
    p = idx0.shape[0] * TOPK
    e = jnp.stack([idx0, idx1], axis=1).reshape(-1)
    onehot = (e[:, None] == jnp.arange(n_exp)[None, :]).astype(jnp.int32)
    cum = jnp.cumsum(onehot, axis=0)
    rank = jnp.sum(onehot * cum, axis=1) - 1
    counts = cum[-1]
    padded_counts = -(-counts // BM) * BM
    padded_end = jnp.cumsum(padded_counts)
    padded_start = padded_end - padded_counts
    dest = jnp.sum(onehot * padded_start[None, :], axis=1) + rank
    token_src = jnp.arange(p, dtype=jnp.int32) // TOPK
    wpair = jnp.stack([w0, w1], axis=1).reshape(-1)
    bstart = jnp.arange(nb, dtype=jnp.int32) * BM
    block_expert = jnp.minimum(
        jnp.sum((bstart[:, None] >= padded_end[None, :]).astype(jnp.int32),
                axis=1), n_exp - 1).astype(jnp.int32)
    return dest.astype(jnp.int32), token_src, wpair, block_expert


def _sc_dispatch(x_flat, token_src, dest, wpair, np_rows):
    n, d = x_flat.shape
    p = token_src.shape[0]
    per_w = p // NW
    chunk = 64
    nch = per_w // chunk
    mesh = plsc.VectorSubcoreMesh(
        core_axis_name="c", subcore_axis_name="s",
        num_cores=SC_CORES, num_subcores=SC_SUBCORES)

    @functools.partial(
        pl.kernel,
        out_type=[
            jax.ShapeDtypeStruct((np_rows, d), jnp.float32),
            jax.ShapeDtypeStruct((np_rows,), jnp.float32),
        ],
        mesh=mesh,
        scratch_types=[
            pltpu.VMEM((chunk,), jnp.int32),
            pltpu.VMEM((chunk,), jnp.int32),
            pltpu.VMEM((chunk, d), jnp.float32),
            pltpu.VMEM((chunk,), jnp.float32),
            pltpu.SemaphoreType.DMA,
            pltpu.SemaphoreType.DMA,
        ],
    )
    def dispatch(x_hbm, tok_hbm, dest_hbm, wpair_hbm, xg_hbm, wpad_hbm,
                 tok_v, dst_v, rows_v, wrow_v, sem_g, sem_s):
        wid = lax.axis_index("s") * SC_CORES + lax.axis_index("c")
        base0 = wid * per_w
        for c in range(nch):
            base = base0 + c * chunk
            pltpu.sync_copy(tok_hbm.at[pl.ds(base, chunk)], tok_v)
            pltpu.sync_copy(dest_hbm.at[pl.ds(base, chunk)], dst_v)
            pltpu.sync_copy(wpair_hbm.at[pl.ds(base, chunk)], wrow_v)
            pltpu.async_copy(x_hbm.at[tok_v], rows_v, sem_g).wait()
            pltpu.async_copy(rows_v, xg_hbm.at[dst_v], sem_s).wait()
            pltpu.async_copy(wrow_v, wpad_hbm.at[dst_v], sem_s).wait()

    return dispatch(x_flat, token_src, dest, wpair)


def _sc_combine(y, dest0, dest1):
    np_rows, d = y.shape
    n = dest0.shape[0]
    per_w = n // NW
    chunk = 32
    nch = per_w // chunk
    mesh = plsc.VectorSubcoreMesh(
        core_axis_name="c", subcore_axis_name="s",
        num_cores=SC_CORES, num_subcores=SC_SUBCORES)

    @functools.partial(
        pl.kernel,
        out_type=jax.ShapeDtypeStruct((n, d), jnp.float32),
        mesh=mesh,
        scratch_types=[
            pltpu.VMEM((chunk,), jnp.int32),
            pltpu.VMEM((chunk,), jnp.int32),
            pltpu.VMEM((chunk, d), jnp.float32),
            pltpu.VMEM((chunk, d), jnp.float32),
            pltpu.SemaphoreType.DMA,
        ],
    )
    def combine(y_hbm, d0_hbm, d1_hbm, out_hbm,
                d0_v, d1_v, rows_a, rows_b, sem):
        wid = lax.axis_index("s") * SC_CORES + lax.axis_index("c")
        base0 = wid * per_w
        nslice = d // 16
        for c in range(nch):
            base = base0 + c * chunk
            pltpu.sync_copy(d0_hbm.at[pl.ds(base, chunk)], d0_v)
            pltpu.sync_copy(d1_hbm.at[pl.ds(base, chunk)], d1_v)
            pltpu.async_copy(y_hbm.at[d0_v], rows_a, sem).wait()
            pltpu.async_copy(y_hbm.at[d1_v], rows_b, sem).wait()
            for r in range(chunk):
                def add_body(v, _):
                    off = v * 16
                    rows_a[r, pl.ds(off, 16)] = (
                        rows_a[r, pl.ds(off, 16)] + rows_b[r, pl.ds(off, 16)])
                    return _
                lax.fori_loop(0, nslice, add_body, 0, unroll=4)
            pltpu.sync_copy(rows_a, out_hbm.at[pl.ds(base, chunk)])

    return combine(y, dest0, dest1)


def _ffn_body(se_ref, xg_ref, w1_ref, w2_ref, wp_ref, y_ref, *, nj):
    j = pl.program_id(1)
    xb = xg_ref[...]
    h = lax.dot_general(xb, w1_ref[0], (((1,), (1,)), ((), ())),
                        preferred_element_type=jnp.float32)
    h = _gelu_exact(h)
    part = lax.dot_general(h, w2_ref[0], (((1,), (1,)), ((), ())),
                           preferred_element_type=jnp.float32)

    @pl.when(j == 0)
    def _():
        y_ref[...] = part

    @pl.when(j > 0)
    def _():
        y_ref[...] += part

    @pl.when(j == nj - 1)
    def _():
        y_ref[...] *= wp_ref[0, 0, :][:, None]


def _ffn(xg, wpad, w1, w2, block_expert):
    np_rows, d = xg.shape
    n_exp, ff, _ = w1.shape
    nb = np_rows // BM
    nj = 4
    bf = ff // nj
    wpad3 = wpad.reshape(nb, 1, BM)
    grid_spec = pltpu.PrefetchScalarGridSpec(
        num_scalar_prefetch=1,
        grid=(nb, nj),
        in_specs=[
            pl.BlockSpec((BM, d), lambda i, j, se: (i, 0)),
            pl.BlockSpec((1, bf, d), lambda i, j, se: (se[i], j, 0)),
            pl.BlockSpec((1, d, bf), lambda i, j, se: (se[i], 0, j)),
            pl.BlockSpec((1, 1, BM), lambda i, j, se: (i, 0, 0)),
        ],
        out_specs=pl.BlockSpec((BM, d), lambda i, j, se: (i, 0)),
    )
    return pl.pallas_call(
        functools.partial(_ffn_body, nj=nj),
        grid_spec=grid_spec,
        out_shape=jax.ShapeDtypeStruct((np_rows, d), jnp.float32),
    )(block_expert, xg, w1, w2, wpad3)


def kernel(x, router_w, w1, w2):
    b, t, d = x.shape
    n_exp, ff, _ = w1.shape
    n = b * t
    x_flat = x.reshape(n, d)
    nb = (n * TOPK) // BM + n_exp
    np_rows = nb * BM

    idx0, idx1, gw0, gw1 = _router(x_flat, router_w)
    dest, token_src, wpair, block_expert = _bookkeeping(
        idx0, idx1, gw0, gw1, n_exp, nb)
    xg, wpad = _sc_dispatch(x_flat, token_src, dest, wpair, np_rows)
    y = _ffn(xg, wpad, w1, w2, block_expert)
    dest0 = dest[0::TOPK]
    dest1 = dest[1::TOPK]
    out = _sc_combine(y, dest0, dest1)
    return out.reshape(b, t, d)
